# Initial kernel scaffold; baseline (speedup 1.0000x reference)
import jax
import jax.numpy as jnp
from jax.experimental import pallas as pl


def kernel(categorical, emb_table, proj_W):
    raise NotImplementedError("write your pallas kernel here")


# speedup vs baseline: 6.9685x; 6.9685x over previous
import functools

import jax
import jax.numpy as jnp
from jax import lax
from jax.experimental import pallas as pl
from jax.experimental.pallas import tpu as pltpu
from jax.experimental.pallas import tpu_sc as plsc

N_CAT = 26
EMB = 16
OUT = 64
BATCH = 16384
CAT_SIZE = 100000
TOTAL = BATCH * N_CAT
NW = 32
PER_W = TOTAL // NW
CHUNK = 128
N_CHUNKS = PER_W // CHUNK

_mesh = plsc.VectorSubcoreMesh(core_axis_name="c", subcore_axis_name="s")


@functools.partial(
    pl.kernel,
    out_type=jax.ShapeDtypeStruct((TOTAL, EMB), jnp.float32),
    mesh=_mesh,
    scratch_types=[
        pltpu.VMEM((PER_W,), jnp.int32),
        pltpu.VMEM((PER_W,), jnp.int32),
        pltpu.VMEM((CHUNK, EMB), jnp.float32),
        pltpu.SemaphoreType.DMA,
    ],
    compiler_params=pltpu.CompilerParams(use_tc_tiling_on_sc=False),
)
def _sc_gather(cat_hbm, table_hbm, out_hbm, cat_v, idx_v, rows_v, sem):
    wid = lax.axis_index("s") * 2 + lax.axis_index("c")
    base = wid * PER_W
    pltpu.sync_copy(cat_hbm.at[pl.ds(base, PER_W)], cat_v)

    def compute_idx(i, carry):
        cat = cat_v[pl.ds(i * 16, 16)]
        pos = i * 16 + lax.iota(jnp.int32, 16)
        feat = lax.rem(pos, N_CAT)
        idx_v[pl.ds(i * 16, 16)] = cat + 1 + feat * CAT_SIZE
        return carry

    lax.fori_loop(0, PER_W // 16, compute_idx, 0)

    def gather_chunk(j, carry):
        idxs = idx_v.at[pl.ds(j * CHUNK, CHUNK)]
        pltpu.async_copy(table_hbm.at[idxs], rows_v, sem).wait()
        pltpu.sync_copy(rows_v, out_hbm.at[pl.ds(base + j * CHUNK, CHUNK)])
        return carry

    lax.fori_loop(0, N_CHUNKS, gather_chunk, 0)


_BM = 2048


def _mm_body(x_ref, w_ref, o_ref):
    o_ref[...] = lax.dot_general(
        x_ref[...], w_ref[...],
        (((1,), (1,)), ((), ())),
        preferred_element_type=jnp.float32,
    )


def _tc_matmul(flat, proj_W):
    return pl.pallas_call(
        _mm_body,
        grid=(BATCH // _BM,),
        in_specs=[
            pl.BlockSpec((_BM, N_CAT * EMB), lambda i: (i, 0)),
            pl.BlockSpec((OUT, N_CAT * EMB), lambda i: (0, 0)),
        ],
        out_specs=pl.BlockSpec((_BM, OUT), lambda i: (i, 0)),
        out_shape=jax.ShapeDtypeStruct((BATCH, OUT), jnp.float32),
    )(flat, proj_W)


def kernel(categorical, emb_table, proj_W):
    cat_flat = categorical.reshape(-1)
    rows = _sc_gather(cat_flat, emb_table)
    flat = rows.reshape(BATCH, N_CAT * EMB)
    return _tc_matmul(flat, proj_W)


# speedup vs baseline: 14.3215x; 2.0552x over previous
import functools

import jax
import jax.numpy as jnp
from jax import lax
from jax.experimental import pallas as pl
from jax.experimental.pallas import tpu as pltpu
from jax.experimental.pallas import tpu_sc as plsc

N_CAT = 26
EMB = 16
OUT = 64
BATCH = 16384
CAT_SIZE = 100000
TROWS = N_CAT * CAT_SIZE + 1
TROWS_PAD = 2600064
TOTAL = BATCH * N_CAT
NW = 32
PER_W = TOTAL // NW
CHUNK = 128
N_CHUNKS = PER_W // CHUNK

SB_ROWS = 1024
N_SB = 2539
TAIL_R0 = N_SB * SB_ROWS

_mesh = plsc.VectorSubcoreMesh(core_axis_name="c", subcore_axis_name="s")


@functools.partial(
    pl.kernel,
    out_type=jax.ShapeDtypeStruct((TROWS_PAD * EMB,), jnp.float32),
    mesh=_mesh,
    scratch_types=[
        pltpu.VMEM((EMB, SB_ROWS), jnp.float32),
        pltpu.VMEM((SB_ROWS * EMB,), jnp.float32),
        pltpu.VMEM((128 * EMB,), jnp.float32),
    ],
    compiler_params=pltpu.CompilerParams(use_tc_tiling_on_sc=True, needs_layout_passes=False),
)
def _sc_transpose(tt_hbm, tail_hbm, out_hbm, buf_v, dst_v, tbuf_v):
    wid = lax.axis_index("s") * 2 + lax.axis_index("c")
    iot = lax.iota(jnp.int32, 16)
    idx_base = iot * EMB

    def sb_body(i, carry):
        sb = wid + i * NW
        r0 = pl.multiple_of(sb * SB_ROWS, SB_ROWS)
        pltpu.sync_copy(tt_hbm.at[:, pl.ds(r0, SB_ROWS)], buf_v)

        def rg_body(rg, c2):
            for c in range(EMB):
                v = buf_v[c, pl.ds(rg * 16, 16)]
                plsc.store_scatter(dst_v, [idx_base + (rg * 256 + c)], v)
            return c2
        lax.fori_loop(0, SB_ROWS // 16, rg_body, 0)

        pltpu.sync_copy(dst_v, out_hbm.at[pl.ds(sb * (SB_ROWS * EMB), SB_ROWS * EMB)])
        return carry

    n_sb_w = (N_SB - wid + NW - 1) // NW
    lax.fori_loop(0, n_sb_w, sb_body, 0)

    @pl.when(wid == NW - 1)
    def _tail():
        pltpu.sync_copy(tail_hbm, tbuf_v)
        pltpu.sync_copy(tbuf_v, out_hbm.at[pl.ds(TAIL_R0 * EMB, 128 * EMB)])


@functools.partial(
    pl.kernel,
    out_type=jax.ShapeDtypeStruct((TOTAL, EMB), jnp.float32),
    mesh=_mesh,
    scratch_types=[
        pltpu.VMEM((PER_W,), jnp.int32),
        pltpu.VMEM((PER_W,), jnp.int32),
        pltpu.VMEM((CHUNK, EMB), jnp.float32),
        pltpu.SemaphoreType.DMA,
    ],
    compiler_params=pltpu.CompilerParams(use_tc_tiling_on_sc=False),
)
def _sc_gather(cat_hbm, table_hbm, out_hbm, cat_v, idx_v, rows_v, sem):
    wid = lax.axis_index("s") * 2 + lax.axis_index("c")
    base = wid * PER_W
    pltpu.sync_copy(cat_hbm.at[pl.ds(base, PER_W)], cat_v)

    def compute_idx(i, carry):
        cat = cat_v[pl.ds(i * 16, 16)]
        pos = i * 16 + lax.iota(jnp.int32, 16)
        feat = lax.rem(pos, N_CAT)
        idx_v[pl.ds(i * 16, 16)] = cat + 1 + feat * CAT_SIZE
        return carry

    lax.fori_loop(0, PER_W // 16, compute_idx, 0)

    def gather_chunk(j, carry):
        idxs = idx_v.at[pl.ds(j * CHUNK, CHUNK)]
        pltpu.async_copy(table_hbm.at[idxs], rows_v, sem).wait()
        pltpu.sync_copy(rows_v, out_hbm.at[pl.ds(base + j * CHUNK, CHUNK)])
        return carry

    lax.fori_loop(0, N_CHUNKS, gather_chunk, 0)


_BM = 2048


def _mm_body(x_ref, w_ref, o_ref):
    o_ref[...] = lax.dot_general(
        x_ref[...], w_ref[...],
        (((1,), (1,)), ((), ())),
        preferred_element_type=jnp.float32,
    )


def _tc_matmul(flat, proj_W):
    return pl.pallas_call(
        _mm_body,
        grid=(BATCH // _BM,),
        in_specs=[
            pl.BlockSpec((_BM, N_CAT * EMB), lambda i: (i, 0)),
            pl.BlockSpec((OUT, N_CAT * EMB), lambda i: (0, 0)),
        ],
        out_specs=pl.BlockSpec((_BM, OUT), lambda i: (i, 0)),
        out_shape=jax.ShapeDtypeStruct((BATCH, OUT), jnp.float32),
    )(flat, proj_W)


def kernel(categorical, emb_table, proj_W):
    tail = jnp.pad(emb_table[TAIL_R0:], ((0, 63), (0, 0))).reshape(-1)
    table_rm = _sc_transpose(emb_table.T, tail)
    table_rm = table_rm.reshape(TROWS_PAD, EMB)
    cat_flat = categorical.reshape(-1)
    rows = _sc_gather(cat_flat, table_rm)
    flat = rows.reshape(BATCH, N_CAT * EMB)
    return _tc_matmul(flat, proj_W)


# speedup vs baseline: 21.2887x; 1.4865x over previous
import functools

import jax
import jax.numpy as jnp
from jax import lax
from jax.experimental import pallas as pl
from jax.experimental.pallas import tpu as pltpu
from jax.experimental.pallas import tpu_sc as plsc

N_CAT = 26
EMB = 16
OUT = 64
BATCH = 16384
CAT_SIZE = 100000
TROWS = N_CAT * CAT_SIZE + 1
TROWS_PAD = 2600064
TOTAL = BATCH * N_CAT
NW = 32
PER_W = TOTAL // NW
CHUNK = 128
N_CHUNKS = PER_W // CHUNK

SB_ROWS = 1024
SB_W = SB_ROWS * EMB
N_SB = 2539
N_SB_UP = 2560
TAIL_R0 = N_SB * SB_ROWS

_mesh = plsc.VectorSubcoreMesh(core_axis_name="c", subcore_axis_name="s")


@functools.partial(
    pl.kernel,
    out_type=jax.ShapeDtypeStruct((TROWS_PAD * EMB,), jnp.float32),
    mesh=_mesh,
    scratch_types=[
        pltpu.VMEM((EMB, SB_ROWS), jnp.float32),
        pltpu.VMEM((EMB, SB_ROWS), jnp.float32),
        pltpu.VMEM((SB_W,), jnp.float32),
        pltpu.VMEM((SB_W,), jnp.float32),
        pltpu.VMEM((128 * EMB,), jnp.float32),
        pltpu.SemaphoreType.DMA,
        pltpu.SemaphoreType.DMA,
        pltpu.SemaphoreType.DMA,
        pltpu.SemaphoreType.DMA,
    ],
    compiler_params=pltpu.CompilerParams(
        use_tc_tiling_on_sc=True, needs_layout_passes=False),
)
def _sc_transpose(tt_hbm, tail_hbm, out_hbm,
                  buf0, buf1, dst0, dst1, tbuf_v,
                  sin0, sin1, sout0, sout1):
    wid = lax.axis_index("s") * 2 + lax.axis_index("c")
    iot = lax.iota(jnp.int32, 16)
    idx_base = iot * EMB
    bufs = (buf0, buf1)
    dsts = (dst0, dst1)
    sins = (sin0, sin1)
    souts = (sout0, sout1)

    def sb_of(i):
        return wid + i * NW

    def start_in(i, p):
        @pl.when(sb_of(i) < N_SB)
        def _():
            r0 = pl.multiple_of(sb_of(i) * SB_ROWS, SB_ROWS)
            pltpu.async_copy(tt_hbm.at[:, pl.ds(r0, SB_ROWS)], bufs[p], sins[p])

    def wait_in(i, p):
        @pl.when(sb_of(i) < N_SB)
        def _():
            pltpu.make_async_copy(tt_hbm.at[:, pl.ds(0, SB_ROWS)],
                                  bufs[p], sins[p]).wait()

    def start_out(i, p):
        @pl.when(sb_of(i) < N_SB)
        def _():
            pltpu.async_copy(
                dsts[p], out_hbm.at[pl.ds(sb_of(i) * SB_W, SB_W)], souts[p])

    def wait_out(i, p):
        @pl.when(sb_of(i) < N_SB)
        def _():
            pltpu.make_async_copy(dsts[p],
                                  out_hbm.at[pl.ds(0, SB_W)], souts[p]).wait()

    def compute(i, p):
        @pl.when(sb_of(i) < N_SB)
        def _():
            buf_v, dst_v = bufs[p], dsts[p]

            def rg_body(rg, c2):
                base = idx_base + rg * (16 * EMB)
                for c in range(EMB):
                    v = buf_v[c, pl.ds(rg * 16, 16)]
                    plsc.store_scatter(dst_v, [base + c], v)
                return c2
            lax.fori_loop(0, SB_ROWS // 16, rg_body, 0)

    start_in(0, 0)

    def pipe_body(i2, c2):
        for b in range(2):
            i = i2 * 2 + b
            p = b
            wait_in(i, p)
            start_in(i + 1, 1 - p)

            @pl.when(i >= 2)
            def _():
                wait_out(i - 2, p)

            compute(i, p)
            start_out(i, p)
        return c2

    lax.fori_loop(0, (N_SB_UP // NW) // 2, pipe_body, 0)
    wait_out(N_SB_UP // NW - 2, 0)
    wait_out(N_SB_UP // NW - 1, 1)

    @pl.when(wid == NW - 1)
    def _tail():
        pltpu.sync_copy(tail_hbm, tbuf_v)
        pltpu.sync_copy(tbuf_v, out_hbm.at[pl.ds(TAIL_R0 * EMB, 128 * EMB)])


@functools.partial(
    pl.kernel,
    out_type=jax.ShapeDtypeStruct((TOTAL, EMB), jnp.float32),
    mesh=_mesh,
    scratch_types=[
        pltpu.VMEM((PER_W,), jnp.int32),
        pltpu.VMEM((PER_W,), jnp.int32),
        pltpu.VMEM((CHUNK, EMB), jnp.float32),
        pltpu.VMEM((CHUNK, EMB), jnp.float32),
        pltpu.SemaphoreType.DMA,
        pltpu.SemaphoreType.DMA,
        pltpu.SemaphoreType.DMA,
        pltpu.SemaphoreType.DMA,
    ],
    compiler_params=pltpu.CompilerParams(use_tc_tiling_on_sc=False),
)
def _sc_gather(cat_hbm, table_hbm, out_hbm, cat_v, idx_v,
               rows0, rows1, sg0, sg1, ss0, ss1):
    wid = lax.axis_index("s") * 2 + lax.axis_index("c")
    base = wid * PER_W
    pltpu.sync_copy(cat_hbm.at[pl.ds(base, PER_W)], cat_v)

    def compute_idx(i, carry):
        cat = cat_v[pl.ds(i * 16, 16)]
        pos = i * 16 + lax.iota(jnp.int32, 16)
        feat = lax.rem(pos, N_CAT)
        idx_v[pl.ds(i * 16, 16)] = cat + 1 + feat * CAT_SIZE
        return carry

    lax.fori_loop(0, PER_W // 16, compute_idx, 0)

    rows = (rows0, rows1)
    sgs = (sg0, sg1)
    sss = (ss0, ss1)

    def start_gather(j, p):
        idxs = idx_v.at[pl.ds(j * CHUNK, CHUNK)]
        pltpu.async_copy(table_hbm.at[idxs], rows[p], sgs[p])

    def wait_gather(p):
        pltpu.make_async_copy(table_hbm.at[pl.ds(0, CHUNK)],
                              rows[p], sgs[p]).wait()

    def start_store(j, p):
        pltpu.async_copy(rows[p],
                         out_hbm.at[pl.ds(base + j * CHUNK, CHUNK)], sss[p])

    def wait_store(p):
        pltpu.make_async_copy(rows[p],
                              out_hbm.at[pl.ds(0, CHUNK)], sss[p]).wait()

    start_gather(0, 0)
    start_gather(1, 1)

    def chunk_body(j2, c2):
        for b in range(2):
            j = j2 * 2 + b
            p = b
            wait_gather(p)
            start_store(j, p)

            @pl.when(j + 2 < N_CHUNKS)
            def _():
                wait_store(p)
                start_gather(j + 2, p)
        return c2

    lax.fori_loop(0, N_CHUNKS // 2, chunk_body, 0)
    wait_store(0)
    wait_store(1)


_BM = 2048


def _mm_body(x_ref, w_ref, o_ref):
    o_ref[...] = lax.dot_general(
        x_ref[...], w_ref[...],
        (((1,), (1,)), ((), ())),
        preferred_element_type=jnp.float32,
    )


def _tc_matmul(flat, proj_W):
    return pl.pallas_call(
        _mm_body,
        grid=(BATCH // _BM,),
        in_specs=[
            pl.BlockSpec((_BM, N_CAT * EMB), lambda i: (i, 0)),
            pl.BlockSpec((OUT, N_CAT * EMB), lambda i: (0, 0)),
        ],
        out_specs=pl.BlockSpec((_BM, OUT), lambda i: (i, 0)),
        out_shape=jax.ShapeDtypeStruct((BATCH, OUT), jnp.float32),
    )(flat, proj_W)


def kernel(categorical, emb_table, proj_W):
    tail = jnp.pad(emb_table[TAIL_R0:], ((0, 63), (0, 0))).reshape(-1)
    table_rm = _sc_transpose(emb_table.T, tail)
    table_rm = table_rm.reshape(TROWS_PAD, EMB)
    cat_flat = categorical.reshape(-1)
    rows = _sc_gather(cat_flat, table_rm)
    flat = rows.reshape(BATCH, N_CAT * EMB)
    return _tc_matmul(flat, proj_W)


# speedup vs baseline: 22.6045x; 1.0618x over previous
import functools

import jax
import jax.numpy as jnp
from jax import lax
from jax.experimental import pallas as pl
from jax.experimental.pallas import tpu as pltpu
from jax.experimental.pallas import tpu_sc as plsc

N_CAT = 26
EMB = 16
OUT = 64
BATCH = 16384
CAT_SIZE = 100000
TROWS = N_CAT * CAT_SIZE + 1
TROWS_PAD = 2600064
TOTAL = BATCH * N_CAT
NW = 32
PER_W = TOTAL // NW
CHUNK = 128
N_CHUNKS = PER_W // CHUNK

SB_ROWS = 1024
SB_W = SB_ROWS * EMB
N_SB = 2539
N_SB_UP = 2560
TAIL_R0 = N_SB * SB_ROWS

_mesh = plsc.VectorSubcoreMesh(core_axis_name="c", subcore_axis_name="s")


@functools.partial(
    pl.kernel,
    out_type=jax.ShapeDtypeStruct((TROWS_PAD * EMB,), jnp.float32),
    mesh=_mesh,
    scratch_types=[
        pltpu.VMEM((EMB, SB_ROWS), jnp.float32),
        pltpu.VMEM((EMB, SB_ROWS), jnp.float32),
        pltpu.VMEM((SB_W,), jnp.float32),
        pltpu.VMEM((SB_W,), jnp.float32),
        pltpu.VMEM((128 * EMB,), jnp.float32),
        pltpu.SemaphoreType.DMA,
        pltpu.SemaphoreType.DMA,
        pltpu.SemaphoreType.DMA,
        pltpu.SemaphoreType.DMA,
    ],
    compiler_params=pltpu.CompilerParams(
        use_tc_tiling_on_sc=True, needs_layout_passes=False,
        disable_bounds_checks=True),
)
def _sc_transpose(tt_hbm, tail_hbm, out_hbm,
                  buf0, buf1, dst0, dst1, tbuf_v,
                  sin0, sin1, sout0, sout1):
    wid = lax.axis_index("s") * 2 + lax.axis_index("c")
    iot = lax.iota(jnp.int32, 16)
    idx_base = iot * EMB
    bufs = (buf0, buf1)
    dsts = (dst0, dst1)
    sins = (sin0, sin1)
    souts = (sout0, sout1)

    def sb_of(i):
        return wid + i * NW

    def start_in(i, p):
        @pl.when(sb_of(i) < N_SB)
        def _():
            r0 = pl.multiple_of(sb_of(i) * SB_ROWS, SB_ROWS)
            pltpu.async_copy(tt_hbm.at[:, pl.ds(r0, SB_ROWS)], bufs[p], sins[p])

    def wait_in(i, p):
        @pl.when(sb_of(i) < N_SB)
        def _():
            pltpu.make_async_copy(tt_hbm.at[:, pl.ds(0, SB_ROWS)],
                                  bufs[p], sins[p]).wait()

    def start_out(i, p):
        @pl.when(sb_of(i) < N_SB)
        def _():
            pltpu.async_copy(
                dsts[p], out_hbm.at[pl.ds(sb_of(i) * SB_W, SB_W)], souts[p])

    def wait_out(i, p):
        @pl.when(sb_of(i) < N_SB)
        def _():
            pltpu.make_async_copy(dsts[p],
                                  out_hbm.at[pl.ds(0, SB_W)], souts[p]).wait()

    def compute(i, p):
        @pl.when(sb_of(i) < N_SB)
        def _():
            buf_v, dst_v = bufs[p], dsts[p]

            def rg_body(rg, c2):
                for u in range(2):
                    rgu = rg * 2 + u
                    dwin = dst_v.at[pl.ds(rgu * (16 * EMB), 16 * EMB)]
                    for c in range(EMB):
                        v = buf_v[c, pl.ds(rgu * 16, 16)]
                        plsc.store_scatter(dwin, [idx_base + c], v)
                return c2
            lax.fori_loop(0, SB_ROWS // 32, rg_body, 0)

    start_in(0, 0)

    def pipe_body(i2, c2):
        for b in range(2):
            i = i2 * 2 + b
            p = b
            wait_in(i, p)
            start_in(i + 1, 1 - p)

            @pl.when(i >= 2)
            def _():
                wait_out(i - 2, p)

            compute(i, p)
            start_out(i, p)
        return c2

    lax.fori_loop(0, (N_SB_UP // NW) // 2, pipe_body, 0)
    wait_out(N_SB_UP // NW - 2, 0)
    wait_out(N_SB_UP // NW - 1, 1)

    @pl.when(wid == NW - 1)
    def _tail():
        pltpu.sync_copy(tail_hbm, tbuf_v)
        pltpu.sync_copy(tbuf_v, out_hbm.at[pl.ds(TAIL_R0 * EMB, 128 * EMB)])


@functools.partial(
    pl.kernel,
    out_type=jax.ShapeDtypeStruct((TOTAL, EMB), jnp.float32),
    mesh=_mesh,
    scratch_types=[
        pltpu.VMEM((PER_W,), jnp.int32),
        pltpu.VMEM((PER_W,), jnp.int32),
        pltpu.VMEM((CHUNK, EMB), jnp.float32),
        pltpu.VMEM((CHUNK, EMB), jnp.float32),
        pltpu.VMEM((CHUNK, EMB), jnp.float32),
        pltpu.VMEM((CHUNK, EMB), jnp.float32),
        pltpu.SemaphoreType.DMA,
        pltpu.SemaphoreType.DMA,
        pltpu.SemaphoreType.DMA,
        pltpu.SemaphoreType.DMA,
        pltpu.SemaphoreType.DMA,
        pltpu.SemaphoreType.DMA,
        pltpu.SemaphoreType.DMA,
        pltpu.SemaphoreType.DMA,
    ],
    compiler_params=pltpu.CompilerParams(
        use_tc_tiling_on_sc=False, disable_bounds_checks=True),
)
def _sc_gather(cat_hbm, table_hbm, out_hbm, cat_v, idx_v,
               rows0, rows1, rows2, rows3,
               sg0, sg1, sg2, sg3, ss0, ss1, ss2, ss3):
    wid = lax.axis_index("s") * 2 + lax.axis_index("c")
    base = wid * PER_W
    pltpu.sync_copy(cat_hbm.at[pl.ds(base, PER_W)], cat_v)

    def compute_idx(i, carry):
        cat = cat_v[pl.ds(i * 16, 16)]
        pos = i * 16 + lax.iota(jnp.int32, 16)
        feat = lax.rem(pos, N_CAT)
        idx_v[pl.ds(i * 16, 16)] = cat + 1 + feat * CAT_SIZE
        return carry

    lax.fori_loop(0, PER_W // 16, compute_idx, 0)

    rows = (rows0, rows1, rows2, rows3)
    sgs = (sg0, sg1, sg2, sg3)
    sss = (ss0, ss1, ss2, ss3)

    def start_gather(j, p):
        idxs = idx_v.at[pl.ds(j * CHUNK, CHUNK)]
        pltpu.async_copy(table_hbm.at[idxs], rows[p], sgs[p])

    def wait_gather(p):
        pltpu.make_async_copy(table_hbm.at[pl.ds(0, CHUNK)],
                              rows[p], sgs[p]).wait()

    def start_store(j, p):
        pltpu.async_copy(rows[p],
                         out_hbm.at[pl.ds(base + j * CHUNK, CHUNK)], sss[p])

    def wait_store(p):
        pltpu.make_async_copy(rows[p],
                              out_hbm.at[pl.ds(0, CHUNK)], sss[p]).wait()

    for p in range(4):
        start_gather(p, p)

    def chunk_body(j4, c2):
        for b in range(4):
            j = j4 * 4 + b
            p = b
            wait_gather(p)
            start_store(j, p)

            @pl.when(j + 4 < N_CHUNKS)
            def _():
                wait_store(p)
                start_gather(j + 4, p)
        return c2

    lax.fori_loop(0, N_CHUNKS // 4, chunk_body, 0)
    for p in range(4):
        wait_store(p)


_BM = 2048


def _mm_body(x_ref, w_ref, o_ref):
    o_ref[...] = lax.dot_general(
        x_ref[...], w_ref[...],
        (((1,), (1,)), ((), ())),
        preferred_element_type=jnp.float32,
    )


def _tc_matmul(flat, proj_W):
    return pl.pallas_call(
        _mm_body,
        grid=(BATCH // _BM,),
        in_specs=[
            pl.BlockSpec((_BM, N_CAT * EMB), lambda i: (i, 0)),
            pl.BlockSpec((OUT, N_CAT * EMB), lambda i: (0, 0)),
        ],
        out_specs=pl.BlockSpec((_BM, OUT), lambda i: (i, 0)),
        out_shape=jax.ShapeDtypeStruct((BATCH, OUT), jnp.float32),
    )(flat, proj_W)


def kernel(categorical, emb_table, proj_W):
    tail = jnp.pad(emb_table[TAIL_R0:], ((0, 63), (0, 0))).reshape(-1)
    table_rm = _sc_transpose(emb_table.T, tail)
    table_rm = table_rm.reshape(TROWS_PAD, EMB)
    cat_flat = categorical.reshape(-1)
    rows = _sc_gather(cat_flat, table_rm)
    flat = rows.reshape(BATCH, N_CAT * EMB)
    return _tc_matmul(flat, proj_W)
